# baseline (speedup 1.0000x reference)
import functools

import jax
import jax.numpy as jnp
from jax import lax
from jax.experimental import pallas as pl
from jax.experimental.pallas import tpu as pltpu
from jax.experimental.pallas import tpu_sc as plsc

N_NODES = 10000
N_EDGES = 320000
C = 128
H2 = 2 * C

NS = 16
CHUNK = 32
IBLK = 40
NB = 16
EDGES_PAD = NS * NB * IBLK * CHUNK
TRASH = N_NODES
OFF = 11000
TBL_ROWS = 2 * OFF
S_ROWS = N_NODES + 8
S_OUT_ROWS = OFF
WTILES = 10
WROWS = N_NODES // WTILES
BN = 1000
NBLK = N_NODES // BN

_mesh = plsc.VectorSubcoreMesh(core_axis_name="c", subcore_axis_name="s")

_f32 = jnp.float32


def _deg_body(dst_hbm, deg_out, idx_v, ones_v, zb_v, deg_sh, sem):
    c = lax.axis_index("c")
    s = lax.axis_index("s")

    @pl.when(c == 0)
    def _():
        def fill(i, carry):
            zb_v[pl.ds(i * 16, 16)] = jnp.zeros((16,), _f32)
            return carry
        lax.fori_loop(0, 64, fill, 0)
        def fill1(i, carry):
            ones_v[pl.ds(i * 16, 16)] = jnp.ones((16,), _f32)
            return carry
        lax.fori_loop(0, CHUNK // 16, fill1, 0)

        @pl.when(s < WTILES)
        def _():
            pltpu.sync_copy(zb_v.at[pl.ds(0, 1000)],
                            deg_sh.at[pl.ds(s * 1000, 1000)])
        plsc.subcore_barrier()

        pltpu.sync_copy(dst_hbm.at[s], idx_v)

        for ib in range(NB):
            def chunk(j, carry):
                pltpu.sync_copy(ones_v, deg_sh.at[idx_v.at[ib, j]], add=True)
                return carry
            lax.fori_loop(0, IBLK, chunk, 0)
        plsc.subcore_barrier()

        @pl.when(s < WTILES)
        def _():
            pltpu.sync_copy(deg_sh.at[pl.ds(s * 1000, 1000)],
                            zb_v.at[pl.ds(0, 1000)])
            pltpu.sync_copy(zb_v.at[pl.ds(0, 1000)],
                            deg_out.at[pl.ds(s * 1000, 1000)])


_deg_call = functools.partial(
    pl.kernel,
    out_type=jax.ShapeDtypeStruct((N_NODES,), _f32),
    mesh=_mesh,
    scratch_types=[
        pltpu.VMEM((NB, IBLK, CHUNK), jnp.int32),
        pltpu.VMEM((CHUNK,), _f32),
        pltpu.VMEM((1024,), _f32),
        pltpu.VMEM_SHARED((S_ROWS,), _f32),
        pltpu.SemaphoreType.DMA,
    ],
)(_deg_body)


def _edge_body(p_tbl, q_tbl, dst_hbm, src_hbm, s_out,
               idx_d, idx_do, idx_s, p_buf, q_buf, s_sh, gsem0, gsem1):
    c = lax.axis_index("c")
    s = lax.axis_index("s")
    gsems = (gsem0, gsem1)
    off = c * OFF

    def zrow(i, carry):
        for jj in range(C // 16):
            p_buf[0, i, pl.ds(jj * 16, 16)] = jnp.zeros((16,), _f32)
        return carry
    lax.fori_loop(0, CHUNK, zrow, 0)

    @pl.when(s < WTILES)
    def _():
        for t in range(31):
            pltpu.sync_copy(p_buf.at[0],
                            s_sh.at[pl.ds(s * WROWS + t * CHUNK, CHUNK)])
        pltpu.sync_copy(p_buf.at[0].at[pl.ds(0, 8)],
                        s_sh.at[pl.ds(s * WROWS + 992, 8)])
    plsc.subcore_barrier()

    def compute(b):
        def row(i, carry2):
            for jj in range(C // 16):
                sl = pl.ds(jj * 16, 16)
                v = p_buf[b, i, sl] + q_buf[b, i, sl]
                p_buf[b, i, sl] = jnp.maximum(v, 0.0)
            return carry2
        lax.fori_loop(0, CHUNK, row, 0)

    def issue(j, b):
        pltpu.async_copy(p_tbl.at[idx_do.at[j]], p_buf.at[b], gsems[b])
        pltpu.async_copy(q_tbl.at[idx_s.at[j]], q_buf.at[b], gsems[b])

    def drain(j, b):
        pltpu.make_async_copy(
            p_tbl.at[idx_do.at[j]], p_buf.at[b], gsems[b]).wait()
        pltpu.make_async_copy(
            q_tbl.at[idx_s.at[j]], q_buf.at[b], gsems[b]).wait()

    for ib in range(NB):
        pltpu.sync_copy(dst_hbm.at[s, ib], idx_d)
        pltpu.sync_copy(src_hbm.at[s, ib], idx_s)

        def addoff(i, carry):
            for jj in range(CHUNK // 16):
                sl = pl.ds(jj * 16, 16)
                idx_do[i, sl] = idx_d[i, sl] + off
                idx_s[i, sl] = idx_s[i, sl] + off
            return carry
        lax.fori_loop(0, IBLK, addoff, 0)

        for b in range(2):
            issue(b, b)

        def body(k, carry):
            for b in range(2):
                j = 2 * k + b
                drain(j, b)
                compute(b)
                pltpu.sync_copy(p_buf.at[b], s_sh.at[idx_d.at[j]], add=True)
                issue(j + 2, b)
            return carry
        lax.fori_loop(0, IBLK // 2 - 1, body, 0)

        for b in range(2):
            j = IBLK - 2 + b
            drain(j, b)
            compute(b)
            pltpu.sync_copy(p_buf.at[b], s_sh.at[idx_d.at[j]], add=True)

    plsc.subcore_barrier()

    @pl.when(s < WTILES)
    def _():
        for t in range(31):
            off_r = s * WROWS + t * CHUNK
            pltpu.sync_copy(s_sh.at[pl.ds(off_r, CHUNK)], p_buf.at[0])
            pltpu.sync_copy(p_buf.at[0], s_out.at[c, pl.ds(off_r, CHUNK)])
        off_r = s * WROWS + 992
        pltpu.sync_copy(s_sh.at[pl.ds(off_r, 8)],
                        p_buf.at[0].at[pl.ds(0, 8)])
        pltpu.sync_copy(p_buf.at[0].at[pl.ds(0, 8)],
                        s_out.at[c, pl.ds(off_r, 8)])


_edge_call = functools.partial(
    pl.kernel,
    out_type=jax.ShapeDtypeStruct((2, S_OUT_ROWS, C), _f32),
    mesh=_mesh,
    scratch_types=[
        pltpu.VMEM((IBLK, CHUNK), jnp.int32),
        pltpu.VMEM((IBLK, CHUNK), jnp.int32),
        pltpu.VMEM((IBLK, CHUNK), jnp.int32),
        pltpu.VMEM((2, CHUNK, C), _f32),
        pltpu.VMEM((2, CHUNK, C), _f32),
        pltpu.VMEM_SHARED((S_ROWS, C), _f32),
        pltpu.SemaphoreType.DMA,
        pltpu.SemaphoreType.DMA,
    ],
)(_edge_body)


def _tbl_body(x_ref, w1a, w1b, b1, p_out, q_out):
    xb = x_ref[...]
    p_out[...] = jnp.dot(xb, w1a[...], preferred_element_type=_f32) + b1[...]
    q_out[...] = jnp.dot(xb, w1b[...], preferred_element_type=_f32)


def _tbl_call(x, w1a, w1b, b1):
    row_spec = pl.BlockSpec((BN, C), lambda i, h: (i, 0))
    tbl_spec = pl.BlockSpec((BN, C), lambda i, h: (h * (OFF // BN) + i, 0))
    out2 = [jax.ShapeDtypeStruct((TBL_ROWS, C), _f32)] * 2
    return pl.pallas_call(
        _tbl_body,
        grid=(NBLK, 2),
        in_specs=[
            row_spec,
            pl.BlockSpec((C, C), lambda i, h: (0, h)),
            pl.BlockSpec((C, C), lambda i, h: (0, h)),
            pl.BlockSpec((1, C), lambda i, h: (0, h)),
        ],
        out_specs=[tbl_spec, tbl_spec],
        out_shape=out2,
    )(x, w1a, w1b, b1)


def _upd_body(has_next, *refs):
    if has_next:
        (h_ref, s_ref, deg_ref, wm2, bm2, wu1a, wu1b, bu1, wu2, bu2,
         w1a_n, w1b_n, b1_n, h_out, p_out, q_out) = refs
    else:
        (h_ref, s_ref, deg_ref, wm2, bm2, wu1a, wu1b, bu1, wu2, bu2,
         h_out) = refs
    hb = h_ref[...]
    sc = jnp.concatenate([s_ref[0], s_ref[1]], axis=-1)
    r = 1.0 / jnp.maximum(deg_ref[...], 1.0)
    sc = sc * r
    agg = jnp.dot(sc, wm2[...], preferred_element_type=_f32) + bm2[...]
    t = jnp.dot(hb, wu1a[...], preferred_element_type=_f32)
    t = t + jnp.dot(agg, wu1b[...], preferred_element_type=_f32) + bu1[...]
    t = jnp.maximum(t, 0.0)
    h2 = jnp.dot(t, wu2[...], preferred_element_type=_f32) + bu2[...]
    h_out[...] = h2
    if has_next:
        p_out[...] = jnp.dot(h2, w1a_n[...],
                             preferred_element_type=_f32) + b1_n[...]
        q_out[...] = jnp.dot(h2, w1b_n[...], preferred_element_type=_f32)


def _upd_call(has_next, h, s, deg, wm2, bm2, wu1a, wu1b, bu1, wu2, bu2,
              w1a_n=None, w1b_n=None, b1_n=None):
    if has_next:
        grid = (NBLK, 2)
        row_spec = pl.BlockSpec((BN, C), lambda i, h: (i, 0))
        full = lambda shape: pl.BlockSpec(
            shape, lambda i, h: tuple(0 for _ in shape))
        s_spec = pl.BlockSpec((2, BN, C), lambda i, h: (0, i, 0))
        deg_spec = pl.BlockSpec((BN, 1), lambda i, h: (i, 0))
        half = lambda: pl.BlockSpec((C, C), lambda i, h: (0, h))
        bhalf = lambda: pl.BlockSpec((1, C), lambda i, h: (0, h))
        tbl_spec = pl.BlockSpec((BN, C), lambda i, h: (h * (OFF // BN) + i, 0))
        in_specs = [row_spec, s_spec, deg_spec,
                    full((H2, C)), full((1, C)),
                    full((C, H2)), full((C, H2)), full((1, H2)),
                    full((H2, C)), full((1, C)),
                    half(), half(), bhalf()]
        args = [h, s, deg, wm2, bm2, wu1a, wu1b, bu1, wu2, bu2,
                w1a_n, w1b_n, b1_n]
        out_specs = [row_spec, tbl_spec, tbl_spec]
        out_shape = [jax.ShapeDtypeStruct((N_NODES, C), _f32),
                     jax.ShapeDtypeStruct((TBL_ROWS, C), _f32),
                     jax.ShapeDtypeStruct((TBL_ROWS, C), _f32)]
    else:
        grid = (NBLK,)
        row_spec = pl.BlockSpec((BN, C), lambda i: (i, 0))
        full = lambda shape: pl.BlockSpec(
            shape, lambda i: tuple(0 for _ in shape))
        s_spec = pl.BlockSpec((2, BN, C), lambda i: (0, i, 0))
        deg_spec = pl.BlockSpec((BN, 1), lambda i: (i, 0))
        in_specs = [row_spec, s_spec, deg_spec,
                    full((H2, C)), full((1, C)),
                    full((C, H2)), full((C, H2)), full((1, H2)),
                    full((H2, C)), full((1, C))]
        args = [h, s, deg, wm2, bm2, wu1a, wu1b, bu1, wu2, bu2]
        out_specs = row_spec
        out_shape = jax.ShapeDtypeStruct((N_NODES, C), _f32)
    return pl.pallas_call(
        functools.partial(_upd_body, has_next),
        grid=grid,
        in_specs=in_specs,
        out_specs=out_specs,
        out_shape=out_shape,
    )(*args)


def kernel(x, edge_index, Wm1, bm1, Wm2, bm2, Wu1, bu1, Wu2, bu2):
    pad = jnp.full((EDGES_PAD - N_EDGES,), TRASH, jnp.int32)
    src = jnp.concatenate([edge_index[0], pad]).reshape(NS, NB, IBLK, CHUNK)
    dst = jnp.concatenate([edge_index[1], pad]).reshape(NS, NB, IBLK, CHUNK)
    n_layers = Wm1.shape[0]

    deg = _deg_call(dst).reshape(N_NODES, 1)

    p_tbl, q_tbl = _tbl_call(x, Wm1[0][:C], Wm1[0][C:], bm1[0].reshape(1, H2))

    h = x
    for l in range(n_layers):
        s = _edge_call(p_tbl, q_tbl, dst, src)
        common = (h, s, deg, Wm2[l], bm2[l].reshape(1, C),
                  Wu1[l][:C], Wu1[l][C:], bu1[l].reshape(1, H2),
                  Wu2[l], bu2[l].reshape(1, C))
        if l + 1 < n_layers:
            h, p_tbl, q_tbl = _upd_call(
                True, *common,
                Wm1[l + 1][:C], Wm1[l + 1][C:], bm1[l + 1].reshape(1, H2))
        else:
            h = _upd_call(False, *common)
    return h


# speedup vs baseline: 1.2091x; 1.2091x over previous
import functools

import jax
import jax.numpy as jnp
from jax import lax
from jax.experimental import pallas as pl
from jax.experimental.pallas import tpu as pltpu
from jax.experimental.pallas import tpu_sc as plsc

N_NODES = 10000
N_EDGES = 320000
C = 128
H2 = 2 * C

NS = 16
CHUNK = 64
IBLK = 40
NB = 8
EDGES_PAD = NS * NB * IBLK * CHUNK
TRASH = N_NODES
OFF = 11000
TBL_ROWS = 4 * OFF
S_ROWS = N_NODES + 8
S_OUT_ROWS = OFF
WTILES = 10
WROWS = N_NODES // WTILES
BN = 1000
NBLK = N_NODES // BN

_mesh = plsc.VectorSubcoreMesh(core_axis_name="c", subcore_axis_name="s")

_f32 = jnp.float32


def _deg_body(dst_hbm, deg_out, idx_v, ones_v, zb_v, deg_sh, sem):
    c = lax.axis_index("c")
    s = lax.axis_index("s")

    @pl.when(c == 0)
    def _():
        def fill(i, carry):
            zb_v[pl.ds(i * 16, 16)] = jnp.zeros((16,), _f32)
            return carry
        lax.fori_loop(0, 64, fill, 0)
        def fill1(i, carry):
            ones_v[pl.ds(i * 16, 16)] = jnp.ones((16,), _f32)
            return carry
        lax.fori_loop(0, CHUNK // 16, fill1, 0)

        @pl.when(s < WTILES)
        def _():
            pltpu.sync_copy(zb_v.at[pl.ds(0, 1000)],
                            deg_sh.at[pl.ds(s * 1000, 1000)])
        plsc.subcore_barrier()

        pltpu.sync_copy(dst_hbm.at[s], idx_v)

        for ib in range(NB):
            def chunk(j, carry):
                pltpu.sync_copy(ones_v, deg_sh.at[idx_v.at[ib, j]], add=True)
                return carry
            lax.fori_loop(0, IBLK, chunk, 0)
        plsc.subcore_barrier()

        @pl.when(s < WTILES)
        def _():
            pltpu.sync_copy(deg_sh.at[pl.ds(s * 1000, 1000)],
                            zb_v.at[pl.ds(0, 1000)])
            pltpu.sync_copy(zb_v.at[pl.ds(0, 1000)],
                            deg_out.at[pl.ds(s * 1000, 1000)])


_deg_call = functools.partial(
    pl.kernel,
    out_type=jax.ShapeDtypeStruct((N_NODES,), _f32),
    mesh=_mesh,
    scratch_types=[
        pltpu.VMEM((NB, IBLK, CHUNK), jnp.int32),
        pltpu.VMEM((CHUNK,), _f32),
        pltpu.VMEM((1024,), _f32),
        pltpu.VMEM_SHARED((S_ROWS,), _f32),
        pltpu.SemaphoreType.DMA,
    ],
)(_deg_body)


def _edge_body(tbl, dst_hbm, src_hbm, s_out,
               idx_d, idx_s, idx_g, gbuf, s_sh, gsem0, gsem1):
    c = lax.axis_index("c")
    s = lax.axis_index("s")
    gsems = (gsem0, gsem1)
    p_off = c * OFF
    q_off = p_off + 2 * OFF

    def zrow(i, carry):
        for jj in range(C // 16):
            gbuf[0, i, pl.ds(jj * 16, 16)] = jnp.zeros((16,), _f32)
        return carry
    lax.fori_loop(0, CHUNK, zrow, 0)

    @pl.when(s < WTILES)
    def _():
        for t in range(15):
            pltpu.sync_copy(gbuf.at[0, pl.ds(0, CHUNK)],
                            s_sh.at[pl.ds(s * WROWS + t * CHUNK, CHUNK)])
        pltpu.sync_copy(gbuf.at[0, pl.ds(0, 40)],
                        s_sh.at[pl.ds(s * WROWS + 960, 40)])
    plsc.subcore_barrier()

    def compute(b):
        def row(i, carry2):
            for jj in range(C // 16):
                sl = pl.ds(jj * 16, 16)
                v = gbuf[b, i, sl] + gbuf[b, CHUNK + i, sl]
                gbuf[b, i, sl] = jnp.maximum(v, 0.0)
            return carry2
        lax.fori_loop(0, CHUNK, row, 0)

    def issue(j, b):
        pltpu.async_copy(tbl.at[idx_g.at[j]], gbuf.at[b], gsems[b])

    def drain(j, b):
        pltpu.make_async_copy(
            tbl.at[idx_g.at[j]], gbuf.at[b], gsems[b]).wait()

    for ib in range(NB):
        pltpu.sync_copy(dst_hbm.at[s, ib], idx_d)
        pltpu.sync_copy(src_hbm.at[s, ib], idx_s)

        def addoff(i, carry):
            for jj in range(CHUNK // 16):
                sl = pl.ds(jj * 16, 16)
                idx_g[i, sl] = idx_d[i, sl] + p_off
                idx_g[i, pl.ds(CHUNK + jj * 16, 16)] = idx_s[i, sl] + q_off
            return carry
        lax.fori_loop(0, IBLK, addoff, 0)

        for b in range(2):
            issue(b, b)

        def body(k, carry):
            for b in range(2):
                j = 2 * k + b
                drain(j, b)
                compute(b)
                pltpu.sync_copy(gbuf.at[b, pl.ds(0, CHUNK)],
                                s_sh.at[idx_d.at[j]], add=True)
                issue(j + 2, b)
            return carry
        lax.fori_loop(0, IBLK // 2 - 1, body, 0)

        for b in range(2):
            j = IBLK - 2 + b
            drain(j, b)
            compute(b)
            pltpu.sync_copy(gbuf.at[b, pl.ds(0, CHUNK)],
                            s_sh.at[idx_d.at[j]], add=True)

    plsc.subcore_barrier()

    @pl.when(s < WTILES)
    def _():
        for t in range(15):
            off_r = s * WROWS + t * CHUNK
            pltpu.sync_copy(s_sh.at[pl.ds(off_r, CHUNK)],
                            gbuf.at[0, pl.ds(0, CHUNK)])
            pltpu.sync_copy(gbuf.at[0, pl.ds(0, CHUNK)],
                            s_out.at[c, pl.ds(off_r, CHUNK)])
        off_r = s * WROWS + 960
        pltpu.sync_copy(s_sh.at[pl.ds(off_r, 40)], gbuf.at[0, pl.ds(0, 40)])
        pltpu.sync_copy(gbuf.at[0, pl.ds(0, 40)],
                        s_out.at[c, pl.ds(off_r, 40)])


_edge_call = functools.partial(
    pl.kernel,
    out_type=jax.ShapeDtypeStruct((2, S_OUT_ROWS, C), _f32),
    mesh=_mesh,
    scratch_types=[
        pltpu.VMEM((IBLK, CHUNK), jnp.int32),
        pltpu.VMEM((IBLK, CHUNK), jnp.int32),
        pltpu.VMEM((IBLK, 2 * CHUNK), jnp.int32),
        pltpu.VMEM((2, 2 * CHUNK, C), _f32),
        pltpu.VMEM_SHARED((S_ROWS, C), _f32),
        pltpu.SemaphoreType.DMA,
        pltpu.SemaphoreType.DMA,
    ],
)(_edge_body)


def _tbl_body(x_ref, w_cat, b_cat, t_out):
    t_out[...] = (jnp.dot(x_ref[...], w_cat[...], preferred_element_type=_f32)
                  + b_cat[...])


def _tbl_call(x, w_cat, b_cat):
    return pl.pallas_call(
        _tbl_body,
        grid=(NBLK, 4),
        in_specs=[
            pl.BlockSpec((BN, C), lambda i, k: (i, 0)),
            pl.BlockSpec((C, C), lambda i, k: (0, k)),
            pl.BlockSpec((1, C), lambda i, k: (0, k)),
        ],
        out_specs=pl.BlockSpec((BN, C), lambda i, k: (k * (OFF // BN) + i, 0)),
        out_shape=jax.ShapeDtypeStruct((TBL_ROWS, C), _f32),
    )(x, w_cat, b_cat)


def _upd_body(has_next, *refs):
    if has_next:
        (h_ref, s_ref, deg_ref, wm2, bm2, wu1a, wu1b, bu1, wu2, bu2,
         w_cat_n, b_cat_n, h_out, t_out) = refs
    else:
        (h_ref, s_ref, deg_ref, wm2, bm2, wu1a, wu1b, bu1, wu2, bu2,
         h_out) = refs
    hb = h_ref[...]
    sc = jnp.concatenate([s_ref[0], s_ref[1]], axis=-1)
    r = 1.0 / jnp.maximum(deg_ref[...], 1.0)
    sc = sc * r
    agg = jnp.dot(sc, wm2[...], preferred_element_type=_f32) + bm2[...]
    t = jnp.dot(hb, wu1a[...], preferred_element_type=_f32)
    t = t + jnp.dot(agg, wu1b[...], preferred_element_type=_f32) + bu1[...]
    t = jnp.maximum(t, 0.0)
    h2 = jnp.dot(t, wu2[...], preferred_element_type=_f32) + bu2[...]
    h_out[...] = h2
    if has_next:
        t_out[...] = (jnp.dot(h2, w_cat_n[...], preferred_element_type=_f32)
                      + b_cat_n[...])


def _upd_call(has_next, h, s, deg, wm2, bm2, wu1a, wu1b, bu1, wu2, bu2,
              w_cat_n=None, b_cat_n=None):
    if has_next:
        grid = (NBLK, 4)
        row_spec = pl.BlockSpec((BN, C), lambda i, k: (i, 0))
        full = lambda shape: pl.BlockSpec(
            shape, lambda i, k: tuple(0 for _ in shape))
        s_spec = pl.BlockSpec((2, BN, C), lambda i, k: (0, i, 0))
        deg_spec = pl.BlockSpec((BN, 1), lambda i, k: (i, 0))
        quarter = pl.BlockSpec((C, C), lambda i, k: (0, k))
        bquarter = pl.BlockSpec((1, C), lambda i, k: (0, k))
        tbl_spec = pl.BlockSpec((BN, C), lambda i, k: (k * (OFF // BN) + i, 0))
        in_specs = [row_spec, s_spec, deg_spec,
                    full((H2, C)), full((1, C)),
                    full((C, H2)), full((C, H2)), full((1, H2)),
                    full((H2, C)), full((1, C)),
                    quarter, bquarter]
        args = [h, s, deg, wm2, bm2, wu1a, wu1b, bu1, wu2, bu2,
                w_cat_n, b_cat_n]
        out_specs = [row_spec, tbl_spec]
        out_shape = [jax.ShapeDtypeStruct((N_NODES, C), _f32),
                     jax.ShapeDtypeStruct((TBL_ROWS, C), _f32)]
    else:
        grid = (NBLK,)
        row_spec = pl.BlockSpec((BN, C), lambda i: (i, 0))
        full = lambda shape: pl.BlockSpec(
            shape, lambda i: tuple(0 for _ in shape))
        s_spec = pl.BlockSpec((2, BN, C), lambda i: (0, i, 0))
        deg_spec = pl.BlockSpec((BN, 1), lambda i: (i, 0))
        in_specs = [row_spec, s_spec, deg_spec,
                    full((H2, C)), full((1, C)),
                    full((C, H2)), full((C, H2)), full((1, H2)),
                    full((H2, C)), full((1, C))]
        args = [h, s, deg, wm2, bm2, wu1a, wu1b, bu1, wu2, bu2]
        out_specs = row_spec
        out_shape = jax.ShapeDtypeStruct((N_NODES, C), _f32)
    return pl.pallas_call(
        functools.partial(_upd_body, has_next),
        grid=grid,
        in_specs=in_specs,
        out_specs=out_specs,
        out_shape=out_shape,
    )(*args)


def kernel(x, edge_index, Wm1, bm1, Wm2, bm2, Wu1, bu1, Wu2, bu2):
    pad = jnp.full((EDGES_PAD - N_EDGES,), TRASH, jnp.int32)
    src = jnp.concatenate([edge_index[0], pad]).reshape(NS, NB, IBLK, CHUNK)
    dst = jnp.concatenate([edge_index[1], pad]).reshape(NS, NB, IBLK, CHUNK)
    n_layers = Wm1.shape[0]

    deg = _deg_call(dst).reshape(N_NODES, 1)

    def cat_w(l):
        return jnp.concatenate([Wm1[l][:C], Wm1[l][C:]], axis=1)

    def cat_b(l):
        return jnp.concatenate([bm1[l], jnp.zeros((H2,), _f32)]).reshape(1, 2 * H2)

    tbl = _tbl_call(x, cat_w(0), cat_b(0))

    h = x
    for l in range(n_layers):
        s = _edge_call(tbl, dst, src)
        common = (h, s, deg, Wm2[l], bm2[l].reshape(1, C),
                  Wu1[l][:C], Wu1[l][C:], bu1[l].reshape(1, H2),
                  Wu2[l], bu2[l].reshape(1, C))
        if l + 1 < n_layers:
            h, tbl = _upd_call(True, *common, cat_w(l + 1), cat_b(l + 1))
        else:
            h = _upd_call(False, *common)
    return h
